# baseline (speedup 1.0000x reference)
import functools

import jax
import jax.numpy as jnp
from jax import lax
from jax.experimental import pallas as pl
from jax.experimental.pallas import tpu as pltpu
from jax.experimental.pallas import tpu_sc as plsc

N = 10000
E = 320000
D_IN = 128
H = 256
NC = 2
NS = 16
CHUNK = 128
NCH = E // CHUNK
ROWS_PER_SUB = 632


@functools.cache
def _sc_conv(feature_split):
  mesh = plsc.VectorSubcoreMesh(core_axis_name="c", subcore_axis_name="s")

  USL = NCH // NS if feature_split else NCH // (NS * NC)
  NB = USL // 6
  REM = USL - NB * 6
  NTAIL = NCH - USL * (NS if feature_split else NS * NC)
  CPS = 1

  def body(edges, table, out, acc,
           ib0, ib1, ib2, rows0, rows1,
           is0, is1, is2, gs0, gs1, ss0, ss1):
    c = lax.axis_index("c")
    s = lax.axis_index("s")
    off = c * N if feature_split else c * 0
    rbase = jnp.minimum(s * ROWS_PER_SUB, N - ROWS_PER_SUB)
    pieces = [(0, 128), (128, 128), (256, 128), (384, 128),
              (512, ROWS_PER_SUB - 512)]
    for o, n in pieces:
      pltpu.sync_copy(table.at[pl.ds(off + rbase + o, n)],
                      rows0.at[pl.ds(0, n)])
      pltpu.sync_copy(rows0.at[pl.ds(0, n)], acc.at[pl.ds(rbase + o, n)])
    plsc.subcore_barrier()

    wid = s if feature_split else s * NC + c
    cbase = wid * USL * CPS
    ib = (ib0, ib1, ib2)
    isem = (is0, is1, is2)
    rows = (rows0, rows1)
    gsem = (gs0, gs1)
    ssem = (ss0, ss1)

    def fire_idx(q, v):
      pltpu.async_copy(edges.at[pl.ds(cbase + CPS * v, CPS)], ib[q], isem[q])

    def wait_idx(q):
      pltpu.make_async_copy(edges.at[pl.ds(0, CPS)], ib[q], isem[q]).wait()
      if feature_split:
        for r in range(CPS):
          for k in range(CHUNK // 16):
            sl = pl.ds(k * 16, 16)
            ib[q][r, 0, sl] = ib[q][r, 0, sl] + off

    def fire_gather(q, p):
      for r in range(CPS):
        pltpu.async_copy(table.at[ib[q].at[r, 0]],
                         rows[p].at[pl.ds(r * CHUNK, CHUNK)], gsem[p])

    def wait_gather(p):
      pltpu.make_async_copy(
          table.at[pl.ds(0, CPS * CHUNK)], rows[p], gsem[p]).wait()

    def fire_scatter(q, p):
      for r in range(CPS):
        pltpu.async_copy(rows[p].at[pl.ds(r * CHUNK, CHUNK)],
                         acc.at[ib[q].at[r, 1]], ssem[p], add=True)

    def wait_scatter(p):
      pltpu.make_async_copy(
          table.at[pl.ds(0, CPS * CHUNK)], rows[p], ssem[p]).wait()

    def slot(t, l):
      static = isinstance(t, int)
      v = 6 * t + l
      q, p = l % 3, l % 2
      wait_gather(p)
      fire_scatter(q, p)
      if static:
        if t > 0 or l > 0:
          wait_scatter(1 - p)
      elif l == 0:
        @pl.when(t > 0)
        def _():
          wait_scatter(1 - p)
      else:
        wait_scatter(1 - p)
      if static:
        if v + 2 < USL:
          fire_idx((l + 2) % 3, v + 2)
      elif REM == 0 and l in (4, 5):
        @pl.when(t < NB - 1)
        def _():
          fire_idx((l + 2) % 3, v + 2)
      else:
        fire_idx((l + 2) % 3, v + 2)
      if static:
        if v + 1 < USL:
          wait_idx((l + 1) % 3)
          fire_gather((l + 1) % 3, 1 - p)
      elif REM == 0 and l == 5:
        @pl.when(t < NB - 1)
        def _():
          wait_idx(0)
          fire_gather(0, 1 - p)
      else:
        wait_idx((l + 1) % 3)
        fire_gather((l + 1) % 3, 1 - p)

    fire_idx(0, 0)
    fire_idx(1, 1)
    wait_idx(0)
    fire_gather(0, 0)

    def block(t, carry):
      for l in range(6):
        slot(t, l)
      return carry

    lax.fori_loop(0, NB, block, 0)
    for l in range(REM):
      slot(NB, l)
    wait_scatter((USL - 1) % 2)

    @pl.when(wid < NTAIL)
    def _():
      cid = NCH - NTAIL + wid
      pltpu.sync_copy(edges.at[pl.ds(cid, 1)], ib0.at[pl.ds(0, 1)])
      if feature_split:
        for k in range(CHUNK // 16):
          sl = pl.ds(k * 16, 16)
          ib0[0, 0, sl] = ib0[0, 0, sl] + off
      pltpu.async_copy(table.at[ib0.at[0, 0]],
                       rows0.at[pl.ds(0, CHUNK)], gs0).wait()
      pltpu.sync_copy(rows0.at[pl.ds(0, CHUNK)], acc.at[ib0.at[0, 1]],
                      add=True)

    plsc.subcore_barrier()
    for o, n in pieces:
      pltpu.sync_copy(acc.at[pl.ds(rbase + o, n)], rows0.at[pl.ds(0, n)])
      pltpu.sync_copy(rows0.at[pl.ds(0, n)],
                      out.at[pl.ds(c * N + rbase + o, n)])

  return pl.kernel(
      body,
      out_type=jax.ShapeDtypeStruct((2 * N, 128), jnp.float32),
      mesh=mesh,
      scratch_types=(
          [pltpu.VMEM_SHARED((N, 128), jnp.float32)]
          + [pltpu.VMEM((1, 2, CHUNK), jnp.int32)] * 3
          + [pltpu.VMEM((CHUNK, 128), jnp.float32)] * 2
          + [pltpu.SemaphoreType.DMA] * 7
      ),
  )


_B = 2000
_G = N // _B


def _mlp1_body(c1_ref, x_ref, w1_ref, b1_ref, w2_ref, b2_ref, o_ref):
  hp = c1_ref[0] + c1_ref[1] - x_ref[...]
  t = jnp.maximum(
      jnp.dot(hp, w1_ref[...], preferred_element_type=jnp.float32)
      + b1_ref[...], 0.0)
  y = jnp.maximum(
      jnp.dot(t, w2_ref[...], preferred_element_type=jnp.float32)
      + b2_ref[...], 0.0)
  o_ref[0] = y[:, :128]
  o_ref[1] = y[:, 128:]


def _mlp_mid_body(hp_ref, w1_ref, b1_ref, w2_ref, b2_ref, o_ref):
  t = jnp.maximum(
      jnp.dot(hp_ref[0], w1_ref[:128], preferred_element_type=jnp.float32)
      + jnp.dot(hp_ref[1], w1_ref[128:], preferred_element_type=jnp.float32)
      + b1_ref[...], 0.0)
  y = jnp.maximum(
      jnp.dot(t, w2_ref[...], preferred_element_type=jnp.float32)
      + b2_ref[...], 0.0)
  o_ref[0] = y[:, :128]
  o_ref[1] = y[:, 128:]


def _attn_body(hp_ref, w1_ref, b1_ref, w2_ref, b2_ref,
               wa_ref, ba_ref, wb_ref, bb_ref, wc_ref, bc_ref,
               wr_ref, br_ref, wcls_ref, bcls_ref,
               logits_ref, hz_ref, s_out_ref,
               m_ref, sum_ref, v_ref):
  i = pl.program_id(0)

  @pl.when(i == 0)
  def _():
    m_ref[0, 0] = -jnp.inf
    sum_ref[0, 0] = 0.0
    v_ref[...] = jnp.zeros_like(v_ref)

  t = jnp.maximum(
      jnp.dot(hp_ref[0], w1_ref[:128], preferred_element_type=jnp.float32)
      + jnp.dot(hp_ref[1], w1_ref[128:], preferred_element_type=jnp.float32)
      + b1_ref[...], 0.0)
  x3 = jnp.maximum(
      jnp.dot(t, w2_ref[...], preferred_element_type=jnp.float32)
      + b2_ref[...], 0.0)
  a = jnp.tanh(
      jnp.dot(x3, wa_ref[...], preferred_element_type=jnp.float32)
      + ba_ref[...])
  b = jax.nn.sigmoid(
      jnp.dot(x3, wb_ref[...], preferred_element_type=jnp.float32)
      + bb_ref[...])
  att = (jnp.dot(a * b, wc_ref[...], preferred_element_type=jnp.float32)
         + bc_ref[...])

  m_old = m_ref[0, 0]
  m_new = jnp.maximum(m_old, jnp.max(att))
  corr = jnp.exp(m_old - m_new)
  w = jnp.exp(att - m_new)
  sum_ref[0, 0] = sum_ref[0, 0] * corr + jnp.sum(w)
  v_ref[...] = v_ref[...] * corr + jnp.sum(w * x3, axis=0, keepdims=True)
  m_ref[0, 0] = m_new

  @pl.when(i == _G - 1)
  def _():
    h = v_ref[...] / sum_ref[0, 0]
    h2 = jnp.maximum(
        jnp.dot(h, wr_ref[...], preferred_element_type=jnp.float32)
        + br_ref[...], 0.0)
    logits = (jnp.dot(h2, wcls_ref[...], preferred_element_type=jnp.float32)
              + bcls_ref[...])
    hz = jax.nn.sigmoid(logits)
    om = 1.0 - hz
    s0 = om[:, 0:1]
    s1 = s0 * om[:, 1:2]
    s2 = s1 * om[:, 2:3]
    s3 = s2 * om[:, 3:4]
    logits_ref[...] = logits
    hz_ref[...] = hz
    s_out_ref[...] = jnp.concatenate([s0, s1, s2, s3], axis=1)


def _full_spec(shape):
  return pl.BlockSpec(shape, lambda i: tuple(0 for _ in shape))


def _mlp1(c1, x, w1, b1, w2, b2):
  return pl.pallas_call(
      _mlp1_body,
      grid=(_G,),
      in_specs=[
          pl.BlockSpec((2, _B, 128), lambda i: (0, i, 0)),
          pl.BlockSpec((_B, 128), lambda i: (i, 0)),
          _full_spec((128, H)),
          _full_spec((1, H)),
          _full_spec((H, H)),
          _full_spec((1, H)),
      ],
      out_specs=pl.BlockSpec((2, _B, 128), lambda i: (0, i, 0)),
      out_shape=jax.ShapeDtypeStruct((2, N, 128), jnp.float32),
      compiler_params=pltpu.CompilerParams(
          dimension_semantics=("arbitrary",)),
  )(c1, x, w1, b1, w2, b2)


def _mlp_mid(hp, w1, b1, w2, b2):
  return pl.pallas_call(
      _mlp_mid_body,
      grid=(_G,),
      in_specs=[
          pl.BlockSpec((2, _B, 128), lambda i: (0, i, 0)),
          _full_spec((H, H)),
          _full_spec((1, H)),
          _full_spec((H, H)),
          _full_spec((1, H)),
      ],
      out_specs=pl.BlockSpec((2, _B, 128), lambda i: (0, i, 0)),
      out_shape=jax.ShapeDtypeStruct((2, N, 128), jnp.float32),
      compiler_params=pltpu.CompilerParams(
          dimension_semantics=("arbitrary",)),
  )(hp, w1, b1, w2, b2)


def _attn(hp, w1, b1, w2, b2, wa, ba, wb, bb, wc, bc, wr, br, wcls, bcls):
  C = wcls.shape[1]
  return pl.pallas_call(
      _attn_body,
      grid=(_G,),
      in_specs=[
          pl.BlockSpec((2, _B, 128), lambda i: (0, i, 0)),
          _full_spec((H, H)), _full_spec((1, H)),
          _full_spec((H, H)), _full_spec((1, H)),
          _full_spec((H, H)), _full_spec((1, H)),
          _full_spec((H, H)), _full_spec((1, H)),
          _full_spec((H, 1)), _full_spec((1, 1)),
          _full_spec((H, H)), _full_spec((1, H)),
          _full_spec((H, C)), _full_spec((1, C)),
      ],
      out_specs=[
          pl.BlockSpec((1, C), lambda i: (0, 0)),
          pl.BlockSpec((1, C), lambda i: (0, 0)),
          pl.BlockSpec((1, C), lambda i: (0, 0)),
      ],
      out_shape=[
          jax.ShapeDtypeStruct((1, C), jnp.float32),
          jax.ShapeDtypeStruct((1, C), jnp.float32),
          jax.ShapeDtypeStruct((1, C), jnp.float32),
      ],
      scratch_shapes=[
          pltpu.SMEM((1, 1), jnp.float32),
          pltpu.SMEM((1, 1), jnp.float32),
          pltpu.VMEM((1, H), jnp.float32),
      ],
      compiler_params=pltpu.CompilerParams(
          dimension_semantics=("arbitrary",)),
  )(hp, w1, b1, w2, b2, wa, ba, wb, bb, wc, bc, wr, br, wcls, bcls)


def kernel(x, edge_index, W11, b11, W12, b12, W21, b21, W22, b22,
           W31, b31, W32, b32, Wa, ba, Wb, bb, Wc, bc, Wr, br, Wcls, bcls):
  edges3 = jnp.stack(
      [edge_index[0].reshape(NCH, CHUNK), edge_index[1].reshape(NCH, CHUNK)],
      axis=1)
  r = lambda v: v.reshape(1, -1)

  c1 = _sc_conv(False)(edges3, x)
  y1 = _mlp1(c1.reshape(2, N, 128), x, W11, r(b11), W12, r(b12))
  c2 = _sc_conv(True)(edges3, y1.reshape(2 * N, 128))
  y2 = _mlp_mid(c2.reshape(2, N, 128), W21, r(b21), W22, r(b22))
  c3 = _sc_conv(True)(edges3, y2.reshape(2 * N, 128))
  logits, hazards, S = _attn(
      c3.reshape(2, N, 128), W31, r(b31), W32, r(b32),
      Wa, r(ba), Wb, r(bb), Wc, r(bc), Wr, r(br), Wcls, r(bcls))
  return (logits, hazards, S)
